# baseline (speedup 1.0000x reference)
import functools

import jax
import jax.numpy as jnp
from jax import lax
from jax.experimental import pallas as pl
from jax.experimental.pallas import tpu as pltpu
from jax.experimental.pallas import tpu_sc as plsc

NC = 2
NS = 16
NW = NC * NS
LANES = 16
CHUNK = 128


def _row_blocks(n):
    blk = 2000
    assert n % blk == 0
    return blk, n // blk


def _prologue_tc(xf, t, s, W_emb, b_emb, Wst_h, Wst_t, Wst_s, b_st, ln_g, ln_b):
    n, H = xf.shape[0], W_emb.shape[1]
    blk, grid = _row_blocks(n)

    def body(x_r, t_r, s_r, we_r, be_r, wh_r, wt_r, ws_r, bst_r, g_r, b_r, o_r):
        h = jnp.dot(x_r[...], we_r[...], preferred_element_type=jnp.float32) + be_r[...]
        h = (jnp.dot(h, wh_r[...], preferred_element_type=jnp.float32)
             + jnp.dot(t_r[...], wt_r[...], preferred_element_type=jnp.float32)
             + jnp.dot(s_r[...], ws_r[...], preferred_element_type=jnp.float32)
             + bst_r[...])
        mu = jnp.mean(h, axis=-1, keepdims=True)
        var = jnp.mean((h - mu) ** 2, axis=-1, keepdims=True)
        h = (h - mu) * lax.rsqrt(var + 1e-5) * g_r[...] + b_r[...]
        o_r[...] = jnp.maximum(h, 0.0)

    full = lambda a: pl.BlockSpec(a.shape, lambda i: (0,) * a.ndim)
    rows = lambda a: pl.BlockSpec((blk,) + a.shape[1:], lambda i: (i,) + (0,) * (a.ndim - 1))
    return pl.pallas_call(
        body,
        grid=(grid,),
        in_specs=[rows(xf), rows(t), rows(s), full(W_emb), full(b_emb),
                  full(Wst_h), full(Wst_t), full(Wst_s), full(b_st),
                  full(ln_g), full(ln_b)],
        out_specs=pl.BlockSpec((blk, H), lambda i: (i, 0)),
        out_shape=jax.ShapeDtypeStruct((n, H), jnp.float32),
    )(xf, t, s, W_emb, b_emb, Wst_h, Wst_t, Wst_s, b_st, ln_g, ln_b)


QWP = 128


def _proj_tc(h, Wq, bq, Wk, bk, Wv, bv, WeT, be2):
    n, H = h.shape
    ED = WeT.shape[1]
    HH = H // 2
    blk, grid = _row_blocks(n)

    def body(h_r, wq_r, bq_r, wk_r, bk_r, wv_r, bv_r, wet_r, be_r,
             q_r, kp_r, vlo_r, vhi_r, qw_r):
        hh = h_r[...]
        q = jnp.dot(hh, wq_r[...], preferred_element_type=jnp.float32) + bq_r[...]
        q_r[...] = q
        kp_r[...] = (jnp.dot(hh, wk_r[...], preferred_element_type=jnp.float32)
                     + bk_r[...] + be_r[...])
        v = (jnp.dot(hh, wv_r[...], preferred_element_type=jnp.float32)
             + bv_r[...] + be_r[...])
        vlo_r[...] = v[:, :HH]
        vhi_r[...] = v[:, HH:]
        qw = jnp.dot(q, wet_r[...], preferred_element_type=jnp.float32)
        qw_r[...] = jnp.concatenate(
            [qw, jnp.zeros((blk, QWP - ED), jnp.float32)], axis=-1)

    full = lambda a: pl.BlockSpec(a.shape, lambda i: (0,) * a.ndim)
    rows2 = lambda d: pl.BlockSpec((blk, d), lambda i: (i, 0))
    return pl.pallas_call(
        body,
        grid=(grid,),
        in_specs=[rows2(H), full(Wq), full(bq), full(Wk), full(bk),
                  full(Wv), full(bv), full(WeT), full(be2)],
        out_specs=[rows2(H), rows2(H), rows2(HH), rows2(HH), rows2(QWP)],
        out_shape=[jax.ShapeDtypeStruct((n, H), jnp.float32),
                   jax.ShapeDtypeStruct((n, H), jnp.float32),
                   jax.ShapeDtypeStruct((n, HH), jnp.float32),
                   jax.ShapeDtypeStruct((n, HH), jnp.float32),
                   jax.ShapeDtypeStruct((n, QWP), jnp.float32)],
    )(h, Wq, bq, Wk, bk, Wv, bv, WeT, be2)


def _epilogue_tc(h, agglo, agghi, sea0, sea1, We, Ws, bs):
    n, H = h.shape
    ED = We.shape[0]
    HH = H // 2
    blk, grid = _row_blocks(n)

    def body(h_r, alo_r, ahi_r, se0_r, se1_r, we_r, ws_r, bs_r, o_r):
        hh = h_r[...]
        se = se0_r[...] + se1_r[...]
        S = se[:, :ED]
        D = se[:, ED:ED + 1]
        Dg = jnp.where(D > 0.0, D, 1.0)
        A = jnp.concatenate([alo_r[...], ahi_r[...]], axis=-1)
        agg = (A + jnp.dot(S, we_r[...], preferred_element_type=jnp.float32)) / Dg
        out = agg + jnp.dot(hh, ws_r[...], preferred_element_type=jnp.float32) + bs_r[...]
        o_r[...] = hh + jnp.maximum(out, 0.0)

    full = lambda a: pl.BlockSpec(a.shape, lambda i: (0,) * a.ndim)
    rows2 = lambda d: pl.BlockSpec((blk, d), lambda i: (i, 0))
    return pl.pallas_call(
        body,
        grid=(grid,),
        in_specs=[rows2(H), rows2(HH), rows2(HH), rows2(2 * ED), rows2(2 * ED),
                  full(We), full(Ws), full(bs)],
        out_specs=rows2(H),
        out_shape=jax.ShapeDtypeStruct((n, H), jnp.float32),
    )(h, agglo, agghi, sea0, sea1, We, Ws, bs)


def _decoder_tc(h, W_dec, b_dec):
    n, H = h.shape
    DO = W_dec.shape[1]
    blk, grid = _row_blocks(n)

    def body(h_r, w_r, b_r, o_r):
        o_r[...] = (jnp.dot(h_r[...], w_r[...], preferred_element_type=jnp.float32)
                    + b_r[...])

    full = lambda a: pl.BlockSpec(a.shape, lambda i: (0,) * a.ndim)
    return pl.pallas_call(
        body,
        grid=(grid,),
        in_specs=[pl.BlockSpec((blk, H), lambda i: (i, 0)), full(W_dec), full(b_dec)],
        out_specs=pl.BlockSpec((blk, DO), lambda i: (i, 0)),
        out_shape=jax.ShapeDtypeStruct((n, DO), jnp.float32),
    )(h, W_dec, b_dec)


def _lane_gather(v, idx16):
    dnums = lax.GatherDimensionNumbers(
        offset_dims=(), collapsed_slice_dims=(0,), start_index_map=(0,))
    return lax.gather(v, idx16[:, None], dnums, slice_sizes=(1,),
                      mode=lax.GatherScatterMode.PROMISE_IN_BOUNDS)


def _bfly_sum(v, lane):
    for sh in (8, 4, 2, 1):
        v = v + _lane_gather(v, lane ^ sh)
    return v


def _bfly_max(v, lane):
    for sh in (8, 4, 2, 1):
        v = jnp.maximum(v, _lane_gather(v, lane ^ sh))
    return v


def _make_logits_sc(N, E, H, ED):
    NCH = E // CHUNK
    CPW = (NCH + NW - 1) // NW
    HV = H // LANES
    scale = 1.0 / float(H) ** 0.5
    mesh = plsc.VectorSubcoreMesh(core_axis_name="c", subcore_axis_name="s")

    @functools.partial(
        pl.kernel,
        out_type=(jax.ShapeDtypeStruct((E,), jnp.float32),
                  jax.ShapeDtypeStruct((NW * LANES,), jnp.float32)),
        mesh=mesh,
        scratch_types=[
            pltpu.VMEM((CHUNK,), jnp.int32),
            pltpu.VMEM((CHUNK,), jnp.int32),
            pltpu.VMEM((CHUNK, H), jnp.float32),
            pltpu.VMEM((CHUNK, H), jnp.float32),
            pltpu.VMEM((CHUNK, QWP), jnp.float32),
            pltpu.VMEM((CHUNK, ED), jnp.float32),
            pltpu.VMEM((CHUNK,), jnp.float32),
            pltpu.VMEM((LANES,), jnp.float32),
            pltpu.SemaphoreType.DMA,
            pltpu.SemaphoreType.DMA,
            pltpu.SemaphoreType.DMA,
        ],
    )
    def kern(q_hbm, kp_hbm, qw_hbm, ea_hbm, src_hbm, dst_hbm,
             logits_hbm, maxes_hbm,
             src_v, dst_v, kbuf, qbuf, qwbuf, eabuf, lbuf, mxbuf, sem0, sem1, sem2):
        c = lax.axis_index("c")
        s = lax.axis_index("s")
        wid = s * NC + c
        lane = lax.iota(jnp.int32, LANES)

        mxbuf[pl.ds(0, LANES)] = jnp.full((LANES,), -jnp.inf, jnp.float32)

        def chunk_body(i, carry):
            g = wid + i * NW

            def do(carry):
                base = g * CHUNK
                pltpu.sync_copy(src_hbm.at[pl.ds(base, CHUNK)], src_v)
                pltpu.sync_copy(dst_hbm.at[pl.ds(base, CHUNK)], dst_v)
                cp0 = pltpu.async_copy(kp_hbm.at[src_v], kbuf, sem0)
                cp1 = pltpu.async_copy(q_hbm.at[dst_v], qbuf, sem1)
                cp2 = pltpu.async_copy(qw_hbm.at[dst_v], qwbuf, sem2)
                pltpu.sync_copy(ea_hbm.at[pl.ds(base, CHUNK), :], eabuf)
                cp0.wait()
                cp1.wait()
                cp2.wait()

                def group_body(gi, carry2):
                    accs = []
                    for t in range(LANES):
                        e = gi * LANES + t
                        acc = qwbuf[e, pl.ds(0, LANES)] * eabuf[e, :]
                        for j in range(HV):
                            acc = acc + (qbuf[e, pl.ds(j * LANES, LANES)]
                                         * kbuf[e, pl.ds(j * LANES, LANES)])
                        accs.append(acc)
                    k = 1
                    while len(accs) > 1:
                        nxt = []
                        for a, b in zip(accs[0::2], accs[1::2]):
                            sa = a + _lane_gather(a, lane ^ k)
                            sb = b + _lane_gather(b, lane ^ k)
                            nxt.append(jnp.where((lane & k) == 0, sa, sb))
                        accs = nxt
                        k *= 2
                    lv = accs[0] * scale
                    lbuf[pl.ds(gi * LANES, LANES)] = lv
                    mxbuf[pl.ds(0, LANES)] = jnp.maximum(mxbuf[pl.ds(0, LANES)], lv)
                    return carry2

                lax.fori_loop(0, CHUNK // LANES, group_body, 0)
                pltpu.sync_copy(lbuf, logits_hbm.at[pl.ds(base, CHUNK)])
                return carry

            return lax.cond(g < NCH, do, lambda cc: cc, carry)

        lax.fori_loop(0, CPW, chunk_body, 0)
        moff = pl.multiple_of(wid * LANES, 8)
        pltpu.sync_copy(mxbuf, maxes_hbm.at[pl.ds(moff, LANES)])

    return kern


def _make_agg_sc(N, E, H, ED):
    NCH = E // CHUNK
    CPS = (NCH + NS - 1) // NS
    HH = H // 2
    HHV = HH // LANES
    BLK = 200
    NB = N // BLK
    NBR = (NB + NS - 1) // NS
    assert N % BLK == 0 and BLK % 8 == 0
    mesh = plsc.VectorSubcoreMesh(core_axis_name="c", subcore_axis_name="s")

    @functools.partial(
        pl.kernel,
        out_type=jax.ShapeDtypeStruct((2 * N, HH), jnp.float32),
        mesh=mesh,
        scratch_types=[
            pltpu.VMEM((CHUNK,), jnp.int32),
            pltpu.VMEM((1, CHUNK), jnp.int32),
            pltpu.VMEM((CHUNK,), jnp.float32),
            pltpu.VMEM((CHUNK,), jnp.float32),
            pltpu.VMEM((CHUNK, 128), jnp.float32),
            pltpu.VMEM((NW * LANES,), jnp.float32),
            pltpu.VMEM((BLK, 128), jnp.float32),
            pltpu.VMEM_SHARED((N, 128), jnp.float32),
            pltpu.SemaphoreType.DMA,
        ],
    )
    def kern(logits_hbm, src_hbm, dst_hbm, maxes_hbm, vstk_hbm,
             agg_hbm,
             src_v, dst_v, lbuf, exbuf, vbuf, mbuf, zbuf, agg_sp, sem0):
        c = lax.axis_index("c")
        s = lax.axis_index("s")
        lane = lax.iota(jnp.int32, LANES)

        def zrow(i, carry):
            zr = zbuf.at[i]
            for j in range(128 // LANES):
                zr[pl.ds(j * LANES, LANES)] = jnp.zeros((LANES,), jnp.float32)
            return carry
        lax.fori_loop(0, BLK, zrow, 0)
        for i in range(NBR):
            b = s + i * NS

            @pl.when(b < NB)
            def _():
                r0 = pl.multiple_of(b * BLK, 8)
                pltpu.sync_copy(zbuf, agg_sp.at[pl.ds(r0, BLK), :])

        pltpu.sync_copy(maxes_hbm, mbuf)
        mv = mbuf[pl.ds(0, LANES)]
        for r in range(1, NW):
            mv = jnp.maximum(mv, mbuf[pl.ds(r * LANES, LANES)])
        Mv = _bfly_max(mv, lane)

        plsc.subcore_barrier()

        def chunk_body(i, carry):
            g = s + i * NS

            def do(carry):
                base = g * CHUNK
                pltpu.sync_copy(src_hbm.at[pl.ds(base, CHUNK)], src_v)
                pltpu.sync_copy(dst_hbm.at[pl.ds(base, CHUNK)], dst_v.at[0])
                pltpu.sync_copy(logits_hbm.at[pl.ds(base, CHUNK)], lbuf)

                off = jnp.zeros((LANES,), jnp.int32) + c * N
                for j in range(CHUNK // LANES):
                    src_v[pl.ds(j * LANES, LANES)] = (
                        src_v[pl.ds(j * LANES, LANES)] + off)
                pltpu.async_copy(vstk_hbm.at[src_v], vbuf, sem0).wait()

                for j in range(CHUNK // LANES):
                    exbuf[pl.ds(j * LANES, LANES)] = jnp.exp(
                        lbuf[pl.ds(j * LANES, LANES)] - Mv)

                def group_body(gi, carry2):
                    ev = exbuf[pl.ds(gi * LANES, LANES)]

                    def lane_body(t, carry3):
                        e = gi * LANES + t
                        av = _lane_gather(ev, jnp.full((LANES,), t, jnp.int32))
                        vrow = vbuf.at[e]
                        for j in range(HHV):
                            vrow[pl.ds(j * LANES, LANES)] = (
                                vrow[pl.ds(j * LANES, LANES)] * av)
                        return carry3

                    lax.fori_loop(0, LANES, lane_body, 0)
                    return carry2

                lax.fori_loop(0, CHUNK // LANES, group_body, 0)

                pltpu.sync_copy(vbuf, agg_sp.at[dst_v.at[0]], add=True)
                return carry

            return lax.cond(g < NCH, do, lambda x: x, carry)

        lax.fori_loop(0, CPS, chunk_body, 0)
        plsc.subcore_barrier()

        for i in range(NBR):
            b = s + i * NS

            @pl.when(b < NB)
            def _():
                r0 = pl.multiple_of(b * BLK, 8)
                pltpu.sync_copy(agg_sp.at[pl.ds(r0, BLK), :], zbuf)
                ro = pl.multiple_of(c * N + b * BLK, 8)
                pltpu.sync_copy(zbuf, agg_hbm.at[pl.ds(ro, BLK), :])

    return kern


def _make_sea_sc(N, E, ED):
    NCH = E // CHUNK
    HCH = NCH // 2
    CPS = (HCH + NS - 1) // NS
    BLK = 80
    NB = N // BLK
    NBR = (NB + NS - 1) // NS
    assert NCH % 2 == 0
    mesh = plsc.VectorSubcoreMesh(core_axis_name="c", subcore_axis_name="s")

    @functools.partial(
        pl.kernel,
        out_type=jax.ShapeDtypeStruct((2 * N, 128), jnp.float32),
        mesh=mesh,
        scratch_types=[
            pltpu.VMEM((1, CHUNK), jnp.int32),
            pltpu.VMEM((CHUNK,), jnp.float32),
            pltpu.VMEM((CHUNK,), jnp.float32),
            pltpu.VMEM((CHUNK, ED), jnp.float32),
            pltpu.VMEM((CHUNK, 128), jnp.float32),
            pltpu.VMEM((NW * LANES,), jnp.float32),
            pltpu.VMEM((BLK, 128), jnp.float32),
            pltpu.VMEM_SHARED((N, 128), jnp.float32),
        ],
    )
    def kern(logits_hbm, dst_hbm, ea_hbm, maxes_hbm, sea_hbm,
             dst_v, lbuf, exbuf, eabuf, seabuf, mbuf, zbuf, sea_sp):
        c = lax.axis_index("c")
        s = lax.axis_index("s")
        lane = lax.iota(jnp.int32, LANES)

        def zrow(i, carry):
            zr = zbuf.at[i]
            for j in range(128 // LANES):
                zr[pl.ds(j * LANES, LANES)] = jnp.zeros((LANES,), jnp.float32)
            return carry
        lax.fori_loop(0, BLK, zrow, 0)

        def zsea(i, carry):
            sr = seabuf.at[i]
            for j in range(128 // LANES):
                sr[pl.ds(j * LANES, LANES)] = jnp.zeros((LANES,), jnp.float32)
            return carry
        lax.fori_loop(0, CHUNK, zsea, 0)

        for i in range(NBR):
            b = s + i * NS

            @pl.when(b < NB)
            def _():
                r0 = pl.multiple_of(b * BLK, 8)
                pltpu.sync_copy(zbuf, sea_sp.at[pl.ds(r0, BLK), :])

        pltpu.sync_copy(maxes_hbm, mbuf)
        mv = mbuf[pl.ds(0, LANES)]
        for r in range(1, NW):
            mv = jnp.maximum(mv, mbuf[pl.ds(r * LANES, LANES)])
        Mv = _bfly_max(mv, lane)

        plsc.subcore_barrier()

        def chunk_body(i, carry):
            gl = s + i * NS
            g = c * HCH + gl

            def do(carry):
                base = g * CHUNK
                pltpu.sync_copy(dst_hbm.at[pl.ds(base, CHUNK)], dst_v.at[0])
                pltpu.sync_copy(logits_hbm.at[pl.ds(base, CHUNK)], lbuf)
                pltpu.sync_copy(ea_hbm.at[pl.ds(base, CHUNK), :], eabuf)

                for j in range(CHUNK // LANES):
                    exbuf[pl.ds(j * LANES, LANES)] = jnp.exp(
                        lbuf[pl.ds(j * LANES, LANES)] - Mv)

                def group_body(gi, carry2):
                    ev = exbuf[pl.ds(gi * LANES, LANES)]

                    def lane_body(t, carry3):
                        e = gi * LANES + t
                        av = _lane_gather(ev, jnp.full((LANES,), t, jnp.int32))
                        srow = seabuf.at[e]
                        srow[pl.ds(0, ED)] = eabuf[e, :] * av
                        srow[pl.ds(ED, ED)] = av
                        return carry3

                    lax.fori_loop(0, LANES, lane_body, 0)
                    return carry2

                lax.fori_loop(0, CHUNK // LANES, group_body, 0)
                pltpu.sync_copy(seabuf, sea_sp.at[dst_v.at[0]], add=True)
                return carry

            return lax.cond(gl < HCH, do, lambda x: x, carry)

        lax.fori_loop(0, CPS, chunk_body, 0)
        plsc.subcore_barrier()

        for i in range(NBR):
            b = s + i * NS

            @pl.when(b < NB)
            def _():
                r0 = pl.multiple_of(b * BLK, 8)
                pltpu.sync_copy(sea_sp.at[pl.ds(r0, BLK), :], zbuf)
                ro = pl.multiple_of(c * N + b * BLK, 8)
                pltpu.sync_copy(zbuf, sea_hbm.at[pl.ds(ro, BLK), :])

    return kern


def kernel(x, edge_index, edge_attr, t, s, W_emb, b_emb, W_st, b_st, ln_g, ln_b,
           Wq, bq, Wk, bk, Wv, bv, We, be, Ws, bs, W_dec, b_dec):
    N = x.shape[0]
    E = edge_index.shape[1]
    H = Wq.shape[-1]
    ED = We.shape[1]
    T_DIM = t.shape[1]
    L = Wq.shape[0]
    OUT = W_dec.shape[1]

    xf = x.reshape(N, -1).astype(jnp.float32)
    src = edge_index[0].astype(jnp.int32)
    dst = edge_index[1].astype(jnp.int32)
    ea = edge_attr.astype(jnp.float32)

    h = _prologue_tc(xf, t, s, W_emb, b_emb.reshape(1, H),
                     W_st[:H], W_st[H:H + T_DIM], W_st[H + T_DIM:],
                     b_st.reshape(1, H), ln_g.reshape(1, H), ln_b.reshape(1, H))

    logits_sc = _make_logits_sc(N, E, H, ED)
    agg_sc = _make_agg_sc(N, E, H, ED)
    sea_sc = _make_sea_sc(N, E, ED)

    for l in range(L):
        q, kp, vlo, vhi, qw = _proj_tc(
            h, Wq[l], bq[l].reshape(1, H), Wk[l], bk[l].reshape(1, H),
            Wv[l], bv[l].reshape(1, H), We[l].T, be[l].reshape(1, H))
        logits, maxes = logits_sc(q, kp, qw, ea, src, dst)
        vstk = jnp.concatenate([vlo, vhi], axis=0)
        agg = agg_sc(logits, src, dst, maxes, vstk)
        agglo, agghi = agg[:N], agg[N:]
        sea2 = sea_sc(logits, dst, ea, maxes)
        sea = (sea2[:N, :2 * ED], sea2[N:, :2 * ED])
        h = _epilogue_tc(h, agglo, agghi, sea[0], sea[1], We[l], Ws[l],
                         bs[l].reshape(1, H))

    o = _decoder_tc(h, W_dec, b_dec.reshape(1, OUT))
    return o.reshape(N, OUT // 6, 6)


# speedup vs baseline: 1.4184x; 1.4184x over previous
import functools

import jax
import jax.numpy as jnp
from jax import lax
from jax.experimental import pallas as pl
from jax.experimental.pallas import tpu as pltpu
from jax.experimental.pallas import tpu_sc as plsc

NC = 2
NS = 16
NW = NC * NS
LANES = 16
CHUNK = 128


def _row_blocks(n):
    blk = 2000
    assert n % blk == 0
    return blk, n // blk


def _prologue_tc(xf, t, s, W_emb, b_emb, Wst_h, Wst_t, Wst_s, b_st, ln_g, ln_b):
    n, H = xf.shape[0], W_emb.shape[1]
    blk, grid = _row_blocks(n)

    def body(x_r, t_r, s_r, we_r, be_r, wh_r, wt_r, ws_r, bst_r, g_r, b_r, o_r):
        h = jnp.dot(x_r[...], we_r[...], preferred_element_type=jnp.float32) + be_r[...]
        h = (jnp.dot(h, wh_r[...], preferred_element_type=jnp.float32)
             + jnp.dot(t_r[...], wt_r[...], preferred_element_type=jnp.float32)
             + jnp.dot(s_r[...], ws_r[...], preferred_element_type=jnp.float32)
             + bst_r[...])
        mu = jnp.mean(h, axis=-1, keepdims=True)
        var = jnp.mean((h - mu) ** 2, axis=-1, keepdims=True)
        h = (h - mu) * lax.rsqrt(var + 1e-5) * g_r[...] + b_r[...]
        o_r[...] = jnp.maximum(h, 0.0)

    full = lambda a: pl.BlockSpec(a.shape, lambda i: (0,) * a.ndim)
    rows = lambda a: pl.BlockSpec((blk,) + a.shape[1:], lambda i: (i,) + (0,) * (a.ndim - 1))
    return pl.pallas_call(
        body,
        grid=(grid,),
        in_specs=[rows(xf), rows(t), rows(s), full(W_emb), full(b_emb),
                  full(Wst_h), full(Wst_t), full(Wst_s), full(b_st),
                  full(ln_g), full(ln_b)],
        out_specs=pl.BlockSpec((blk, H), lambda i: (i, 0)),
        out_shape=jax.ShapeDtypeStruct((n, H), jnp.float32),
    )(xf, t, s, W_emb, b_emb, Wst_h, Wst_t, Wst_s, b_st, ln_g, ln_b)


QWP = 128


def _proj_tc(h, Wq, bq, Wk, bk, Wv, bv, WeT, be2):
    n, H = h.shape
    ED = WeT.shape[1]
    HH = H // 2
    blk, grid = _row_blocks(n)

    def body(h_r, wq_r, bq_r, wk_r, bk_r, wv_r, bv_r, wet_r, be_r,
             q_r, kp_r, vlo_r, vhi_r, qw_r):
        hh = h_r[...]
        q = jnp.dot(hh, wq_r[...], preferred_element_type=jnp.float32) + bq_r[...]
        q_r[...] = q
        kp_r[...] = (jnp.dot(hh, wk_r[...], preferred_element_type=jnp.float32)
                     + bk_r[...] + be_r[...])
        v = (jnp.dot(hh, wv_r[...], preferred_element_type=jnp.float32)
             + bv_r[...] + be_r[...])
        vlo_r[...] = v[:, :HH]
        vhi_r[...] = v[:, HH:]
        qw = jnp.dot(q, wet_r[...], preferred_element_type=jnp.float32)
        qw_r[...] = jnp.concatenate(
            [qw, jnp.zeros((blk, QWP - ED), jnp.float32)], axis=-1)

    full = lambda a: pl.BlockSpec(a.shape, lambda i: (0,) * a.ndim)
    rows2 = lambda d: pl.BlockSpec((blk, d), lambda i: (i, 0))
    return pl.pallas_call(
        body,
        grid=(grid,),
        in_specs=[rows2(H), full(Wq), full(bq), full(Wk), full(bk),
                  full(Wv), full(bv), full(WeT), full(be2)],
        out_specs=[rows2(H), rows2(H), rows2(HH), rows2(HH), rows2(QWP)],
        out_shape=[jax.ShapeDtypeStruct((n, H), jnp.float32),
                   jax.ShapeDtypeStruct((n, H), jnp.float32),
                   jax.ShapeDtypeStruct((n, HH), jnp.float32),
                   jax.ShapeDtypeStruct((n, HH), jnp.float32),
                   jax.ShapeDtypeStruct((n, QWP), jnp.float32)],
    )(h, Wq, bq, Wk, bk, Wv, bv, WeT, be2)


def _epilogue_tc(h, agglo, agghi, sea0, sea1, We, Ws, bs):
    n, H = h.shape
    ED = We.shape[0]
    HH = H // 2
    blk, grid = _row_blocks(n)

    def body(h_r, alo_r, ahi_r, se0_r, se1_r, we_r, ws_r, bs_r, o_r):
        hh = h_r[...]
        se = se0_r[...] + se1_r[...]
        S = se[:, :ED]
        D = se[:, ED:ED + 1]
        Dg = jnp.where(D > 0.0, D, 1.0)
        A = jnp.concatenate([alo_r[...], ahi_r[...]], axis=-1)
        agg = (A + jnp.dot(S, we_r[...], preferred_element_type=jnp.float32)) / Dg
        out = agg + jnp.dot(hh, ws_r[...], preferred_element_type=jnp.float32) + bs_r[...]
        o_r[...] = hh + jnp.maximum(out, 0.0)

    full = lambda a: pl.BlockSpec(a.shape, lambda i: (0,) * a.ndim)
    rows2 = lambda d: pl.BlockSpec((blk, d), lambda i: (i, 0))
    return pl.pallas_call(
        body,
        grid=(grid,),
        in_specs=[rows2(H), rows2(HH), rows2(HH), rows2(2 * ED), rows2(2 * ED),
                  full(We), full(Ws), full(bs)],
        out_specs=rows2(H),
        out_shape=jax.ShapeDtypeStruct((n, H), jnp.float32),
    )(h, agglo, agghi, sea0, sea1, We, Ws, bs)


def _decoder_tc(h, W_dec, b_dec):
    n, H = h.shape
    DO = W_dec.shape[1]
    blk, grid = _row_blocks(n)

    def body(h_r, w_r, b_r, o_r):
        o_r[...] = (jnp.dot(h_r[...], w_r[...], preferred_element_type=jnp.float32)
                    + b_r[...])

    full = lambda a: pl.BlockSpec(a.shape, lambda i: (0,) * a.ndim)
    return pl.pallas_call(
        body,
        grid=(grid,),
        in_specs=[pl.BlockSpec((blk, H), lambda i: (i, 0)), full(W_dec), full(b_dec)],
        out_specs=pl.BlockSpec((blk, DO), lambda i: (i, 0)),
        out_shape=jax.ShapeDtypeStruct((n, DO), jnp.float32),
    )(h, W_dec, b_dec)


def _lane_gather(v, idx16):
    dnums = lax.GatherDimensionNumbers(
        offset_dims=(), collapsed_slice_dims=(0,), start_index_map=(0,))
    return lax.gather(v, idx16[:, None], dnums, slice_sizes=(1,),
                      mode=lax.GatherScatterMode.PROMISE_IN_BOUNDS)


def _bfly_sum(v, lane):
    for sh in (8, 4, 2, 1):
        v = v + _lane_gather(v, lane ^ sh)
    return v


def _bfly_max(v, lane):
    for sh in (8, 4, 2, 1):
        v = jnp.maximum(v, _lane_gather(v, lane ^ sh))
    return v


def _make_logits_sc(N, E, H, ED):
    CH = 64
    NCH = E // CH
    CPW = (NCH + NW - 1) // NW
    HV = H // LANES
    scale = 1.0 / float(H) ** 0.5
    mesh = plsc.VectorSubcoreMesh(core_axis_name="c", subcore_axis_name="s")

    @functools.partial(
        pl.kernel,
        out_type=(jax.ShapeDtypeStruct((E,), jnp.float32),
                  jax.ShapeDtypeStruct((NW * LANES,), jnp.float32)),
        mesh=mesh,
        scratch_types=[
            pltpu.VMEM((2, CH), jnp.int32),
            pltpu.VMEM((2, CH), jnp.int32),
            pltpu.VMEM((2, CH, H), jnp.float32),
            pltpu.VMEM((2, CH, H), jnp.float32),
            pltpu.VMEM((2, CH, QWP), jnp.float32),
            pltpu.VMEM((2, CH, ED), jnp.float32),
            pltpu.VMEM((CH,), jnp.float32),
            pltpu.VMEM((LANES,), jnp.float32),
            pltpu.SemaphoreType.DMA,
            pltpu.SemaphoreType.DMA,
        ],
    )
    def kern(q_hbm, kp_hbm, qw_hbm, ea_hbm, src_hbm, dst_hbm,
             logits_hbm, maxes_hbm,
             src_v, dst_v, kbuf, qbuf, qwbuf, eabuf, lbuf, mxbuf, sem0, sem1):
        c = lax.axis_index("c")
        s = lax.axis_index("s")
        wid = s * NC + c
        lane = lax.iota(jnp.int32, LANES)
        sems = (sem0, sem1)

        mxbuf[pl.ds(0, LANES)] = jnp.full((LANES,), -jnp.inf, jnp.float32)

        def enqueue(i, b):
            g = wid + i * NW

            @pl.when(g < NCH)
            def _():
                base = g * CH
                pltpu.sync_copy(src_hbm.at[pl.ds(base, CH)], src_v.at[b])
                pltpu.sync_copy(dst_hbm.at[pl.ds(base, CH)], dst_v.at[b])
                pltpu.async_copy(kp_hbm.at[src_v.at[b]], kbuf.at[b], sems[b])
                pltpu.async_copy(q_hbm.at[dst_v.at[b]], qbuf.at[b], sems[b])
                pltpu.async_copy(qw_hbm.at[dst_v.at[b]], qwbuf.at[b], sems[b])
                pltpu.async_copy(ea_hbm.at[pl.ds(base, CH), :], eabuf.at[b],
                                 sems[b])

        def consume(i, b):
            g = wid + i * NW

            @pl.when(g < NCH)
            def _():
                base = g * CH
                pltpu.make_async_copy(kp_hbm.at[src_v.at[b]], kbuf.at[b],
                                      sems[b]).wait()
                pltpu.make_async_copy(q_hbm.at[dst_v.at[b]], qbuf.at[b],
                                      sems[b]).wait()
                pltpu.make_async_copy(qw_hbm.at[dst_v.at[b]], qwbuf.at[b],
                                      sems[b]).wait()
                pltpu.make_async_copy(ea_hbm.at[pl.ds(base, CH), :],
                                      eabuf.at[b], sems[b]).wait()

                def group_body(gi, carry2):
                    def lane_body(t, res):
                        e = gi * LANES + t
                        acc = qwbuf[b, e, pl.ds(0, LANES)] * eabuf[b, e, :]
                        for j in range(HV):
                            acc = acc + (qbuf[b, e, pl.ds(j * LANES, LANES)]
                                         * kbuf[b, e, pl.ds(j * LANES, LANES)])
                        tot = _bfly_sum(acc, lane) * scale
                        sel = lane == jnp.full((LANES,), t, jnp.int32)
                        return res + jnp.where(
                            sel, tot, jnp.zeros((LANES,), jnp.float32))

                    lv = lax.fori_loop(0, LANES, lane_body,
                                       jnp.zeros((LANES,), jnp.float32))
                    lbuf[pl.ds(gi * LANES, LANES)] = lv
                    mxbuf[pl.ds(0, LANES)] = jnp.maximum(
                        mxbuf[pl.ds(0, LANES)], lv)
                    return carry2

                lax.fori_loop(0, CH // LANES, group_body, 0)
                pltpu.sync_copy(lbuf, logits_hbm.at[pl.ds(base, CH)])

        enqueue(0, 0)

        def pair_body(i2, carry):
            for b in range(2):
                i = i2 * 2 + b
                enqueue(i + 1, 1 - b)
                consume(i, b)
            return carry

        assert CPW % 2 == 0 or True
        NP = (CPW + 1) // 2
        lax.fori_loop(0, NP, pair_body, 0)
        consume(NP * 2, 0)

        moff = pl.multiple_of(wid * LANES, 8)
        pltpu.sync_copy(mxbuf, maxes_hbm.at[pl.ds(moff, LANES)])

    return kern


def _make_agg_sc(N, E, H, ED):
    NCH = E // CHUNK
    CPS = (NCH + NS - 1) // NS
    HH = H // 2
    HHV = HH // LANES
    BLK = 200
    NB = N // BLK
    NBR = (NB + NS - 1) // NS
    assert N % BLK == 0 and BLK % 8 == 0
    mesh = plsc.VectorSubcoreMesh(core_axis_name="c", subcore_axis_name="s")

    @functools.partial(
        pl.kernel,
        out_type=jax.ShapeDtypeStruct((2 * N, HH), jnp.float32),
        mesh=mesh,
        scratch_types=[
            pltpu.VMEM((CHUNK,), jnp.int32),
            pltpu.VMEM((1, CHUNK), jnp.int32),
            pltpu.VMEM((CHUNK,), jnp.float32),
            pltpu.VMEM((CHUNK,), jnp.float32),
            pltpu.VMEM((CHUNK, 128), jnp.float32),
            pltpu.VMEM((NW * LANES,), jnp.float32),
            pltpu.VMEM((BLK, 128), jnp.float32),
            pltpu.VMEM_SHARED((N, 128), jnp.float32),
            pltpu.SemaphoreType.DMA,
        ],
    )
    def kern(logits_hbm, src_hbm, dst_hbm, maxes_hbm, vstk_hbm,
             agg_hbm,
             src_v, dst_v, lbuf, exbuf, vbuf, mbuf, zbuf, agg_sp, sem0):
        c = lax.axis_index("c")
        s = lax.axis_index("s")
        lane = lax.iota(jnp.int32, LANES)

        def zrow(i, carry):
            zr = zbuf.at[i]
            for j in range(128 // LANES):
                zr[pl.ds(j * LANES, LANES)] = jnp.zeros((LANES,), jnp.float32)
            return carry
        lax.fori_loop(0, BLK, zrow, 0)
        for i in range(NBR):
            b = s + i * NS

            @pl.when(b < NB)
            def _():
                r0 = pl.multiple_of(b * BLK, 8)
                pltpu.sync_copy(zbuf, agg_sp.at[pl.ds(r0, BLK), :])

        pltpu.sync_copy(maxes_hbm, mbuf)
        mv = mbuf[pl.ds(0, LANES)]
        for r in range(1, NW):
            mv = jnp.maximum(mv, mbuf[pl.ds(r * LANES, LANES)])
        Mv = _bfly_max(mv, lane)

        plsc.subcore_barrier()

        def chunk_body(i, carry):
            g = s + i * NS

            def do(carry):
                base = g * CHUNK
                pltpu.sync_copy(src_hbm.at[pl.ds(base, CHUNK)], src_v)
                pltpu.sync_copy(dst_hbm.at[pl.ds(base, CHUNK)], dst_v.at[0])
                pltpu.sync_copy(logits_hbm.at[pl.ds(base, CHUNK)], lbuf)

                off = jnp.zeros((LANES,), jnp.int32) + c * N
                for j in range(CHUNK // LANES):
                    src_v[pl.ds(j * LANES, LANES)] = (
                        src_v[pl.ds(j * LANES, LANES)] + off)
                pltpu.async_copy(vstk_hbm.at[src_v], vbuf, sem0).wait()

                for j in range(CHUNK // LANES):
                    exbuf[pl.ds(j * LANES, LANES)] = jnp.exp(
                        lbuf[pl.ds(j * LANES, LANES)] - Mv)

                def group_body(gi, carry2):
                    ev = exbuf[pl.ds(gi * LANES, LANES)]

                    def lane_body(t, carry3):
                        e = gi * LANES + t
                        av = _lane_gather(ev, jnp.full((LANES,), t, jnp.int32))
                        vrow = vbuf.at[e]
                        for j in range(HHV):
                            vrow[pl.ds(j * LANES, LANES)] = (
                                vrow[pl.ds(j * LANES, LANES)] * av)
                        return carry3

                    lax.fori_loop(0, LANES, lane_body, 0)
                    return carry2

                lax.fori_loop(0, CHUNK // LANES, group_body, 0)

                pltpu.sync_copy(vbuf, agg_sp.at[dst_v.at[0]], add=True)
                return carry

            return lax.cond(g < NCH, do, lambda x: x, carry)

        lax.fori_loop(0, CPS, chunk_body, 0)
        plsc.subcore_barrier()

        for i in range(NBR):
            b = s + i * NS

            @pl.when(b < NB)
            def _():
                r0 = pl.multiple_of(b * BLK, 8)
                pltpu.sync_copy(agg_sp.at[pl.ds(r0, BLK), :], zbuf)
                ro = pl.multiple_of(c * N + b * BLK, 8)
                pltpu.sync_copy(zbuf, agg_hbm.at[pl.ds(ro, BLK), :])

    return kern


def _make_sea_sc(N, E, ED):
    NCH = E // CHUNK
    HCH = NCH // 2
    CPS = (HCH + NS - 1) // NS
    BLK = 80
    NB = N // BLK
    NBR = (NB + NS - 1) // NS
    assert NCH % 2 == 0
    mesh = plsc.VectorSubcoreMesh(core_axis_name="c", subcore_axis_name="s")

    @functools.partial(
        pl.kernel,
        out_type=jax.ShapeDtypeStruct((2 * N, 128), jnp.float32),
        mesh=mesh,
        scratch_types=[
            pltpu.VMEM((1, CHUNK), jnp.int32),
            pltpu.VMEM((CHUNK,), jnp.float32),
            pltpu.VMEM((CHUNK,), jnp.float32),
            pltpu.VMEM((CHUNK, ED), jnp.float32),
            pltpu.VMEM((CHUNK, 128), jnp.float32),
            pltpu.VMEM((NW * LANES,), jnp.float32),
            pltpu.VMEM((BLK, 128), jnp.float32),
            pltpu.VMEM_SHARED((N, 128), jnp.float32),
        ],
    )
    def kern(logits_hbm, dst_hbm, ea_hbm, maxes_hbm, sea_hbm,
             dst_v, lbuf, exbuf, eabuf, seabuf, mbuf, zbuf, sea_sp):
        c = lax.axis_index("c")
        s = lax.axis_index("s")
        lane = lax.iota(jnp.int32, LANES)

        def zrow(i, carry):
            zr = zbuf.at[i]
            for j in range(128 // LANES):
                zr[pl.ds(j * LANES, LANES)] = jnp.zeros((LANES,), jnp.float32)
            return carry
        lax.fori_loop(0, BLK, zrow, 0)

        def zsea(i, carry):
            sr = seabuf.at[i]
            for j in range(128 // LANES):
                sr[pl.ds(j * LANES, LANES)] = jnp.zeros((LANES,), jnp.float32)
            return carry
        lax.fori_loop(0, CHUNK, zsea, 0)

        for i in range(NBR):
            b = s + i * NS

            @pl.when(b < NB)
            def _():
                r0 = pl.multiple_of(b * BLK, 8)
                pltpu.sync_copy(zbuf, sea_sp.at[pl.ds(r0, BLK), :])

        pltpu.sync_copy(maxes_hbm, mbuf)
        mv = mbuf[pl.ds(0, LANES)]
        for r in range(1, NW):
            mv = jnp.maximum(mv, mbuf[pl.ds(r * LANES, LANES)])
        Mv = _bfly_max(mv, lane)

        plsc.subcore_barrier()

        def chunk_body(i, carry):
            gl = s + i * NS
            g = c * HCH + gl

            def do(carry):
                base = g * CHUNK
                pltpu.sync_copy(dst_hbm.at[pl.ds(base, CHUNK)], dst_v.at[0])
                pltpu.sync_copy(logits_hbm.at[pl.ds(base, CHUNK)], lbuf)
                pltpu.sync_copy(ea_hbm.at[pl.ds(base, CHUNK), :], eabuf)

                for j in range(CHUNK // LANES):
                    exbuf[pl.ds(j * LANES, LANES)] = jnp.exp(
                        lbuf[pl.ds(j * LANES, LANES)] - Mv)

                def group_body(gi, carry2):
                    ev = exbuf[pl.ds(gi * LANES, LANES)]

                    def lane_body(t, carry3):
                        e = gi * LANES + t
                        av = _lane_gather(ev, jnp.full((LANES,), t, jnp.int32))
                        srow = seabuf.at[e]
                        srow[pl.ds(0, ED)] = eabuf[e, :] * av
                        srow[pl.ds(ED, ED)] = av
                        return carry3

                    lax.fori_loop(0, LANES, lane_body, 0)
                    return carry2

                lax.fori_loop(0, CHUNK // LANES, group_body, 0)
                pltpu.sync_copy(seabuf, sea_sp.at[dst_v.at[0]], add=True)
                return carry

            return lax.cond(gl < HCH, do, lambda x: x, carry)

        lax.fori_loop(0, CPS, chunk_body, 0)
        plsc.subcore_barrier()

        for i in range(NBR):
            b = s + i * NS

            @pl.when(b < NB)
            def _():
                r0 = pl.multiple_of(b * BLK, 8)
                pltpu.sync_copy(sea_sp.at[pl.ds(r0, BLK), :], zbuf)
                ro = pl.multiple_of(c * N + b * BLK, 8)
                pltpu.sync_copy(zbuf, sea_hbm.at[pl.ds(ro, BLK), :])

    return kern


def kernel(x, edge_index, edge_attr, t, s, W_emb, b_emb, W_st, b_st, ln_g, ln_b,
           Wq, bq, Wk, bk, Wv, bv, We, be, Ws, bs, W_dec, b_dec):
    N = x.shape[0]
    E = edge_index.shape[1]
    H = Wq.shape[-1]
    ED = We.shape[1]
    T_DIM = t.shape[1]
    L = Wq.shape[0]
    OUT = W_dec.shape[1]

    xf = x.reshape(N, -1).astype(jnp.float32)
    src = edge_index[0].astype(jnp.int32)
    dst = edge_index[1].astype(jnp.int32)
    ea = edge_attr.astype(jnp.float32)

    h = _prologue_tc(xf, t, s, W_emb, b_emb.reshape(1, H),
                     W_st[:H], W_st[H:H + T_DIM], W_st[H + T_DIM:],
                     b_st.reshape(1, H), ln_g.reshape(1, H), ln_b.reshape(1, H))

    logits_sc = _make_logits_sc(N, E, H, ED)
    agg_sc = _make_agg_sc(N, E, H, ED)
    sea_sc = _make_sea_sc(N, E, ED)

    for l in range(L):
        q, kp, vlo, vhi, qw = _proj_tc(
            h, Wq[l], bq[l].reshape(1, H), Wk[l], bk[l].reshape(1, H),
            Wv[l], bv[l].reshape(1, H), We[l].T, be[l].reshape(1, H))
        logits, maxes = logits_sc(q, kp, qw, ea, src, dst)
        vstk = jnp.concatenate([vlo, vhi], axis=0)
        agg = agg_sc(logits, src, dst, maxes, vstk)
        agglo, agghi = agg[:N], agg[N:]
        sea2 = sea_sc(logits, dst, ea, maxes)
        sea = (sea2[:N, :2 * ED], sea2[N:, :2 * ED])
        h = _epilogue_tc(h, agglo, agghi, sea[0], sea[1], We[l], Ws[l],
                         bs[l].reshape(1, H))

    o = _decoder_tc(h, W_dec, b_dec.reshape(1, OUT))
    return o.reshape(N, OUT // 6, 6)


# speedup vs baseline: 1.4356x; 1.0121x over previous
import functools

import jax
import jax.numpy as jnp
from jax import lax
from jax.experimental import pallas as pl
from jax.experimental.pallas import tpu as pltpu
from jax.experimental.pallas import tpu_sc as plsc

NC = 2
NS = 16
NW = NC * NS
LANES = 16
CHUNK = 128


def _row_blocks(n):
    blk = 2000
    assert n % blk == 0
    return blk, n // blk


def _prologue_tc(xf, t, s, W_emb, b_emb, Wst_h, Wst_t, Wst_s, b_st, ln_g, ln_b):
    n, H = xf.shape[0], W_emb.shape[1]
    blk, grid = _row_blocks(n)

    def body(x_r, t_r, s_r, we_r, be_r, wh_r, wt_r, ws_r, bst_r, g_r, b_r, o_r):
        h = jnp.dot(x_r[...], we_r[...], preferred_element_type=jnp.float32) + be_r[...]
        h = (jnp.dot(h, wh_r[...], preferred_element_type=jnp.float32)
             + jnp.dot(t_r[...], wt_r[...], preferred_element_type=jnp.float32)
             + jnp.dot(s_r[...], ws_r[...], preferred_element_type=jnp.float32)
             + bst_r[...])
        mu = jnp.mean(h, axis=-1, keepdims=True)
        var = jnp.mean((h - mu) ** 2, axis=-1, keepdims=True)
        h = (h - mu) * lax.rsqrt(var + 1e-5) * g_r[...] + b_r[...]
        o_r[...] = jnp.maximum(h, 0.0)

    full = lambda a: pl.BlockSpec(a.shape, lambda i: (0,) * a.ndim)
    rows = lambda a: pl.BlockSpec((blk,) + a.shape[1:], lambda i: (i,) + (0,) * (a.ndim - 1))
    return pl.pallas_call(
        body,
        grid=(grid,),
        in_specs=[rows(xf), rows(t), rows(s), full(W_emb), full(b_emb),
                  full(Wst_h), full(Wst_t), full(Wst_s), full(b_st),
                  full(ln_g), full(ln_b)],
        out_specs=pl.BlockSpec((blk, H), lambda i: (i, 0)),
        out_shape=jax.ShapeDtypeStruct((n, H), jnp.float32),
    )(xf, t, s, W_emb, b_emb, Wst_h, Wst_t, Wst_s, b_st, ln_g, ln_b)


QWP = 128


def _proj_tc(h, Wq, bq, Wk, bk, Wv, bv, WeT, be2):
    n, H = h.shape
    ED = WeT.shape[1]
    HH = H // 2
    blk, grid = _row_blocks(n)

    def body(h_r, wq_r, bq_r, wk_r, bk_r, wv_r, bv_r, wet_r, be_r,
             q_r, kp_r, vlo_r, vhi_r, qw_r):
        hh = h_r[...]
        q = jnp.dot(hh, wq_r[...], preferred_element_type=jnp.float32) + bq_r[...]
        q_r[...] = q
        kp_r[...] = (jnp.dot(hh, wk_r[...], preferred_element_type=jnp.float32)
                     + bk_r[...] + be_r[...])
        v = (jnp.dot(hh, wv_r[...], preferred_element_type=jnp.float32)
             + bv_r[...] + be_r[...])
        vlo_r[...] = v[:, :HH]
        vhi_r[...] = v[:, HH:]
        qw = jnp.dot(q, wet_r[...], preferred_element_type=jnp.float32)
        qw_r[...] = jnp.concatenate(
            [qw, jnp.zeros((blk, QWP - ED), jnp.float32)], axis=-1)

    full = lambda a: pl.BlockSpec(a.shape, lambda i: (0,) * a.ndim)
    rows2 = lambda d: pl.BlockSpec((blk, d), lambda i: (i, 0))
    return pl.pallas_call(
        body,
        grid=(grid,),
        in_specs=[rows2(H), full(Wq), full(bq), full(Wk), full(bk),
                  full(Wv), full(bv), full(WeT), full(be2)],
        out_specs=[rows2(H), rows2(H), rows2(HH), rows2(HH), rows2(QWP)],
        out_shape=[jax.ShapeDtypeStruct((n, H), jnp.float32),
                   jax.ShapeDtypeStruct((n, H), jnp.float32),
                   jax.ShapeDtypeStruct((n, HH), jnp.float32),
                   jax.ShapeDtypeStruct((n, HH), jnp.float32),
                   jax.ShapeDtypeStruct((n, QWP), jnp.float32)],
    )(h, Wq, bq, Wk, bk, Wv, bv, WeT, be2)


def _epilogue_tc(h, agglo, agghi, sea0, sea1, We, Ws, bs):
    n, H = h.shape
    ED = We.shape[0]
    HH = H // 2
    blk, grid = _row_blocks(n)

    def body(h_r, alo_r, ahi_r, se0_r, se1_r, we_r, ws_r, bs_r, o_r):
        hh = h_r[...]
        se = se0_r[...] + se1_r[...]
        S = se[:, :ED]
        D = se[:, ED:ED + 1]
        Dg = jnp.where(D > 0.0, D, 1.0)
        A = jnp.concatenate([alo_r[...], ahi_r[...]], axis=-1)
        agg = (A + jnp.dot(S, we_r[...], preferred_element_type=jnp.float32)) / Dg
        out = agg + jnp.dot(hh, ws_r[...], preferred_element_type=jnp.float32) + bs_r[...]
        o_r[...] = hh + jnp.maximum(out, 0.0)

    full = lambda a: pl.BlockSpec(a.shape, lambda i: (0,) * a.ndim)
    rows2 = lambda d: pl.BlockSpec((blk, d), lambda i: (i, 0))
    return pl.pallas_call(
        body,
        grid=(grid,),
        in_specs=[rows2(H), rows2(HH), rows2(HH), rows2(2 * ED), rows2(2 * ED),
                  full(We), full(Ws), full(bs)],
        out_specs=rows2(H),
        out_shape=jax.ShapeDtypeStruct((n, H), jnp.float32),
    )(h, agglo, agghi, sea0, sea1, We, Ws, bs)


def _decoder_tc(h, W_dec, b_dec):
    n, H = h.shape
    DO = W_dec.shape[1]
    blk, grid = _row_blocks(n)

    def body(h_r, w_r, b_r, o_r):
        o_r[...] = (jnp.dot(h_r[...], w_r[...], preferred_element_type=jnp.float32)
                    + b_r[...])

    full = lambda a: pl.BlockSpec(a.shape, lambda i: (0,) * a.ndim)
    return pl.pallas_call(
        body,
        grid=(grid,),
        in_specs=[pl.BlockSpec((blk, H), lambda i: (i, 0)), full(W_dec), full(b_dec)],
        out_specs=pl.BlockSpec((blk, DO), lambda i: (i, 0)),
        out_shape=jax.ShapeDtypeStruct((n, DO), jnp.float32),
    )(h, W_dec, b_dec)


def _lane_gather(v, idx16):
    dnums = lax.GatherDimensionNumbers(
        offset_dims=(), collapsed_slice_dims=(0,), start_index_map=(0,))
    return lax.gather(v, idx16[:, None], dnums, slice_sizes=(1,),
                      mode=lax.GatherScatterMode.PROMISE_IN_BOUNDS)


def _bfly_sum(v, lane):
    for sh in (8, 4, 2, 1):
        v = v + _lane_gather(v, lane ^ sh)
    return v


def _bfly_max(v, lane):
    for sh in (8, 4, 2, 1):
        v = jnp.maximum(v, _lane_gather(v, lane ^ sh))
    return v


def _make_logits_sc(N, E, H, ED):
    CH = 64
    NCH = E // CH
    CPW = (NCH + NW - 1) // NW
    HV = H // LANES
    scale = 1.0 / float(H) ** 0.5
    mesh = plsc.VectorSubcoreMesh(core_axis_name="c", subcore_axis_name="s")

    @functools.partial(
        pl.kernel,
        out_type=(jax.ShapeDtypeStruct((E,), jnp.float32),
                  jax.ShapeDtypeStruct((NW * LANES,), jnp.float32)),
        mesh=mesh,
        scratch_types=[
            pltpu.VMEM((2, CH), jnp.int32),
            pltpu.VMEM((2, CH), jnp.int32),
            pltpu.VMEM((2, CH, H), jnp.float32),
            pltpu.VMEM((2, CH, H), jnp.float32),
            pltpu.VMEM((2, CH, QWP), jnp.float32),
            pltpu.VMEM((2, CH, ED), jnp.float32),
            pltpu.VMEM((CH,), jnp.float32),
            pltpu.VMEM((LANES,), jnp.float32),
            pltpu.SemaphoreType.DMA,
            pltpu.SemaphoreType.DMA,
        ],
    )
    def kern(q_hbm, kp_hbm, qw_hbm, ea_hbm, src_hbm, dst_hbm,
             logits_hbm, maxes_hbm,
             src_v, dst_v, kbuf, qbuf, qwbuf, eabuf, lbuf, mxbuf, sem0, sem1):
        c = lax.axis_index("c")
        s = lax.axis_index("s")
        wid = s * NC + c
        lane = lax.iota(jnp.int32, LANES)
        sems = (sem0, sem1)

        mxbuf[pl.ds(0, LANES)] = jnp.full((LANES,), -jnp.inf, jnp.float32)

        def enqueue(i, b):
            g = wid + i * NW

            @pl.when(g < NCH)
            def _():
                base = g * CH
                pltpu.sync_copy(src_hbm.at[pl.ds(base, CH)], src_v.at[b])
                pltpu.sync_copy(dst_hbm.at[pl.ds(base, CH)], dst_v.at[b])
                pltpu.async_copy(kp_hbm.at[src_v.at[b]], kbuf.at[b], sems[b])
                pltpu.async_copy(q_hbm.at[dst_v.at[b]], qbuf.at[b], sems[b])
                pltpu.async_copy(qw_hbm.at[dst_v.at[b]], qwbuf.at[b], sems[b])
                pltpu.async_copy(ea_hbm.at[pl.ds(base, CH), :], eabuf.at[b],
                                 sems[b])

        def consume(i, b):
            g = wid + i * NW

            @pl.when(g < NCH)
            def _():
                base = g * CH
                pltpu.make_async_copy(kp_hbm.at[src_v.at[b]], kbuf.at[b],
                                      sems[b]).wait()
                pltpu.make_async_copy(q_hbm.at[dst_v.at[b]], qbuf.at[b],
                                      sems[b]).wait()
                pltpu.make_async_copy(qw_hbm.at[dst_v.at[b]], qwbuf.at[b],
                                      sems[b]).wait()
                pltpu.make_async_copy(ea_hbm.at[pl.ds(base, CH), :],
                                      eabuf.at[b], sems[b]).wait()

                def group_body(gi, carry2):
                    def lane_body(t, res):
                        e = gi * LANES + t
                        acc = qwbuf[b, e, pl.ds(0, LANES)] * eabuf[b, e, :]
                        for j in range(HV):
                            acc = acc + (qbuf[b, e, pl.ds(j * LANES, LANES)]
                                         * kbuf[b, e, pl.ds(j * LANES, LANES)])
                        tot = _bfly_sum(acc, lane) * scale
                        sel = lane == jnp.full((LANES,), t, jnp.int32)
                        return res + jnp.where(
                            sel, tot, jnp.zeros((LANES,), jnp.float32))

                    lv = lax.fori_loop(0, LANES, lane_body,
                                       jnp.zeros((LANES,), jnp.float32))
                    lbuf[pl.ds(gi * LANES, LANES)] = lv
                    mxbuf[pl.ds(0, LANES)] = jnp.maximum(
                        mxbuf[pl.ds(0, LANES)], lv)
                    return carry2

                lax.fori_loop(0, CH // LANES, group_body, 0)
                pltpu.sync_copy(lbuf, logits_hbm.at[pl.ds(base, CH)])

        enqueue(0, 0)

        def pair_body(i2, carry):
            for b in range(2):
                i = i2 * 2 + b
                enqueue(i + 1, 1 - b)
                consume(i, b)
            return carry

        assert CPW % 2 == 0 or True
        NP = (CPW + 1) // 2
        lax.fori_loop(0, NP, pair_body, 0)
        consume(NP * 2, 0)

        moff = pl.multiple_of(wid * LANES, 8)
        pltpu.sync_copy(mxbuf, maxes_hbm.at[pl.ds(moff, LANES)])

    return kern


def _make_agg_sc(N, E, H, ED):
    CH = 64
    NCH = E // CH
    CPS = (NCH + NS - 1) // NS
    HH = H // 2
    HHV = HH // LANES
    BLK = 80
    NB = N // BLK
    NBR = (NB + NS - 1) // NS
    assert N % BLK == 0 and BLK % 8 == 0
    mesh = plsc.VectorSubcoreMesh(core_axis_name="c", subcore_axis_name="s")

    @functools.partial(
        pl.kernel,
        out_type=jax.ShapeDtypeStruct((2 * N, HH), jnp.float32),
        mesh=mesh,
        scratch_types=[
            pltpu.VMEM((2, CH), jnp.int32),
            pltpu.VMEM((2, CH), jnp.int32),
            pltpu.VMEM((2, CH), jnp.float32),
            pltpu.VMEM((CH,), jnp.float32),
            pltpu.VMEM((2, CH, 128), jnp.float32),
            pltpu.VMEM((NW * LANES,), jnp.float32),
            pltpu.VMEM((BLK, 128), jnp.float32),
            pltpu.VMEM_SHARED((N, 128), jnp.float32),
            pltpu.SemaphoreType.DMA,
            pltpu.SemaphoreType.DMA,
        ],
    )
    def kern(logits_hbm, src_hbm, dst_hbm, maxes_hbm, vstk_hbm,
             agg_hbm,
             src_v, dst_v, lbuf, exbuf, vbuf, mbuf, zbuf, agg_sp, sem0, sem1):
        c = lax.axis_index("c")
        s = lax.axis_index("s")
        lane = lax.iota(jnp.int32, LANES)
        sems = (sem0, sem1)

        def zrow(i, carry):
            zr = zbuf.at[i]
            for j in range(128 // LANES):
                zr[pl.ds(j * LANES, LANES)] = jnp.zeros((LANES,), jnp.float32)
            return carry
        lax.fori_loop(0, BLK, zrow, 0)
        for i in range(NBR):
            b = s + i * NS

            @pl.when(b < NB)
            def _():
                r0 = pl.multiple_of(b * BLK, 8)
                pltpu.sync_copy(zbuf, agg_sp.at[pl.ds(r0, BLK), :])

        pltpu.sync_copy(maxes_hbm, mbuf)
        mv = mbuf[pl.ds(0, LANES)]
        for r in range(1, NW):
            mv = jnp.maximum(mv, mbuf[pl.ds(r * LANES, LANES)])
        Mv = _bfly_max(mv, lane)

        plsc.subcore_barrier()

        def enqueue(i, b):
            g = s + i * NS

            @pl.when(g < NCH)
            def _():
                base = g * CH
                pltpu.sync_copy(src_hbm.at[pl.ds(base, CH)], src_v.at[b])
                pltpu.sync_copy(dst_hbm.at[pl.ds(base, CH)], dst_v.at[b])
                pltpu.sync_copy(logits_hbm.at[pl.ds(base, CH)], lbuf.at[b])
                off = jnp.zeros((LANES,), jnp.int32) + c * N
                sb = src_v.at[b]
                for j in range(CH // LANES):
                    sb[pl.ds(j * LANES, LANES)] = sb[pl.ds(j * LANES, LANES)] + off
                pltpu.async_copy(vstk_hbm.at[src_v.at[b]], vbuf.at[b], sems[b])

        def consume(i, b):
            g = s + i * NS

            @pl.when(g < NCH)
            def _():
                pltpu.make_async_copy(vstk_hbm.at[src_v.at[b]], vbuf.at[b],
                                      sems[b]).wait()
                for j in range(CH // LANES):
                    exbuf[pl.ds(j * LANES, LANES)] = jnp.exp(
                        lbuf[b, pl.ds(j * LANES, LANES)] - Mv)

                def group_body(gi, carry2):
                    ev = exbuf[pl.ds(gi * LANES, LANES)]

                    def lane_body(t, carry3):
                        e = gi * LANES + t
                        av = _lane_gather(ev, jnp.full((LANES,), t, jnp.int32))
                        vrow = vbuf.at[b, e]
                        for j in range(HHV):
                            vrow[pl.ds(j * LANES, LANES)] = (
                                vrow[pl.ds(j * LANES, LANES)] * av)
                        return carry3

                    lax.fori_loop(0, LANES, lane_body, 0)
                    return carry2

                lax.fori_loop(0, CH // LANES, group_body, 0)
                pltpu.sync_copy(vbuf.at[b], agg_sp.at[dst_v.at[b]], add=True)

        enqueue(0, 0)

        def pair_body(i2, carry):
            for b in range(2):
                i = i2 * 2 + b
                enqueue(i + 1, 1 - b)
                consume(i, b)
            return carry

        NP = (CPS + 1) // 2
        lax.fori_loop(0, NP, pair_body, 0)
        consume(NP * 2, 0)
        plsc.subcore_barrier()

        for i in range(NBR):
            b = s + i * NS

            @pl.when(b < NB)
            def _():
                r0 = pl.multiple_of(b * BLK, 8)
                pltpu.sync_copy(agg_sp.at[pl.ds(r0, BLK), :], zbuf)
                ro = pl.multiple_of(c * N + b * BLK, 8)
                pltpu.sync_copy(zbuf, agg_hbm.at[pl.ds(ro, BLK), :])

    return kern


def _make_sea_sc(N, E, ED):
    NCH = E // CHUNK
    HCH = NCH // 2
    CPS = (HCH + NS - 1) // NS
    BLK = 80
    NB = N // BLK
    NBR = (NB + NS - 1) // NS
    assert NCH % 2 == 0
    mesh = plsc.VectorSubcoreMesh(core_axis_name="c", subcore_axis_name="s")

    @functools.partial(
        pl.kernel,
        out_type=jax.ShapeDtypeStruct((2 * N, 128), jnp.float32),
        mesh=mesh,
        scratch_types=[
            pltpu.VMEM((1, CHUNK), jnp.int32),
            pltpu.VMEM((CHUNK,), jnp.float32),
            pltpu.VMEM((CHUNK,), jnp.float32),
            pltpu.VMEM((CHUNK, ED), jnp.float32),
            pltpu.VMEM((CHUNK, 128), jnp.float32),
            pltpu.VMEM((NW * LANES,), jnp.float32),
            pltpu.VMEM((BLK, 128), jnp.float32),
            pltpu.VMEM_SHARED((N, 128), jnp.float32),
        ],
    )
    def kern(logits_hbm, dst_hbm, ea_hbm, maxes_hbm, sea_hbm,
             dst_v, lbuf, exbuf, eabuf, seabuf, mbuf, zbuf, sea_sp):
        c = lax.axis_index("c")
        s = lax.axis_index("s")
        lane = lax.iota(jnp.int32, LANES)

        def zrow(i, carry):
            zr = zbuf.at[i]
            for j in range(128 // LANES):
                zr[pl.ds(j * LANES, LANES)] = jnp.zeros((LANES,), jnp.float32)
            return carry
        lax.fori_loop(0, BLK, zrow, 0)

        def zsea(i, carry):
            sr = seabuf.at[i]
            for j in range(128 // LANES):
                sr[pl.ds(j * LANES, LANES)] = jnp.zeros((LANES,), jnp.float32)
            return carry
        lax.fori_loop(0, CHUNK, zsea, 0)

        for i in range(NBR):
            b = s + i * NS

            @pl.when(b < NB)
            def _():
                r0 = pl.multiple_of(b * BLK, 8)
                pltpu.sync_copy(zbuf, sea_sp.at[pl.ds(r0, BLK), :])

        pltpu.sync_copy(maxes_hbm, mbuf)
        mv = mbuf[pl.ds(0, LANES)]
        for r in range(1, NW):
            mv = jnp.maximum(mv, mbuf[pl.ds(r * LANES, LANES)])
        Mv = _bfly_max(mv, lane)

        plsc.subcore_barrier()

        def chunk_body(i, carry):
            gl = s + i * NS
            g = c * HCH + gl

            def do(carry):
                base = g * CHUNK
                pltpu.sync_copy(dst_hbm.at[pl.ds(base, CHUNK)], dst_v.at[0])
                pltpu.sync_copy(logits_hbm.at[pl.ds(base, CHUNK)], lbuf)
                pltpu.sync_copy(ea_hbm.at[pl.ds(base, CHUNK), :], eabuf)

                for j in range(CHUNK // LANES):
                    exbuf[pl.ds(j * LANES, LANES)] = jnp.exp(
                        lbuf[pl.ds(j * LANES, LANES)] - Mv)

                def group_body(gi, carry2):
                    ev = exbuf[pl.ds(gi * LANES, LANES)]

                    def lane_body(t, carry3):
                        e = gi * LANES + t
                        av = _lane_gather(ev, jnp.full((LANES,), t, jnp.int32))
                        srow = seabuf.at[e]
                        srow[pl.ds(0, ED)] = eabuf[e, :] * av
                        srow[pl.ds(ED, ED)] = av
                        return carry3

                    lax.fori_loop(0, LANES, lane_body, 0)
                    return carry2

                lax.fori_loop(0, CHUNK // LANES, group_body, 0)
                pltpu.sync_copy(seabuf, sea_sp.at[dst_v.at[0]], add=True)
                return carry

            return lax.cond(gl < HCH, do, lambda x: x, carry)

        lax.fori_loop(0, CPS, chunk_body, 0)
        plsc.subcore_barrier()

        for i in range(NBR):
            b = s + i * NS

            @pl.when(b < NB)
            def _():
                r0 = pl.multiple_of(b * BLK, 8)
                pltpu.sync_copy(sea_sp.at[pl.ds(r0, BLK), :], zbuf)
                ro = pl.multiple_of(c * N + b * BLK, 8)
                pltpu.sync_copy(zbuf, sea_hbm.at[pl.ds(ro, BLK), :])

    return kern


def kernel(x, edge_index, edge_attr, t, s, W_emb, b_emb, W_st, b_st, ln_g, ln_b,
           Wq, bq, Wk, bk, Wv, bv, We, be, Ws, bs, W_dec, b_dec):
    N = x.shape[0]
    E = edge_index.shape[1]
    H = Wq.shape[-1]
    ED = We.shape[1]
    T_DIM = t.shape[1]
    L = Wq.shape[0]
    OUT = W_dec.shape[1]

    xf = x.reshape(N, -1).astype(jnp.float32)
    src = edge_index[0].astype(jnp.int32)
    dst = edge_index[1].astype(jnp.int32)
    ea = edge_attr.astype(jnp.float32)

    h = _prologue_tc(xf, t, s, W_emb, b_emb.reshape(1, H),
                     W_st[:H], W_st[H:H + T_DIM], W_st[H + T_DIM:],
                     b_st.reshape(1, H), ln_g.reshape(1, H), ln_b.reshape(1, H))

    logits_sc = _make_logits_sc(N, E, H, ED)
    agg_sc = _make_agg_sc(N, E, H, ED)
    sea_sc = _make_sea_sc(N, E, ED)

    for l in range(L):
        q, kp, vlo, vhi, qw = _proj_tc(
            h, Wq[l], bq[l].reshape(1, H), Wk[l], bk[l].reshape(1, H),
            Wv[l], bv[l].reshape(1, H), We[l].T, be[l].reshape(1, H))
        logits, maxes = logits_sc(q, kp, qw, ea, src, dst)
        vstk = jnp.concatenate([vlo, vhi], axis=0)
        agg = agg_sc(logits, src, dst, maxes, vstk)
        agglo, agghi = agg[:N], agg[N:]
        sea2 = sea_sc(logits, dst, ea, maxes)
        sea = (sea2[:N, :2 * ED], sea2[N:, :2 * ED])
        h = _epilogue_tc(h, agglo, agghi, sea[0], sea[1], We[l], Ws[l],
                         bs[l].reshape(1, H))

    o = _decoder_tc(h, W_dec, b_dec.reshape(1, OUT))
    return o.reshape(N, OUT // 6, 6)
